# baseline (speedup 1.0000x reference)
import functools

import jax
import jax.numpy as jnp
from jax.experimental import pallas as pl
from jax.experimental.pallas import tpu as pltpu

N_NODES = 50000
K_NB = 31
D = 50
D_ENT = 128
DP = 64
BLK = 400


def _body(q_ref, ent_ref, para_ref, score_ref, WentT_ref, bentP_ref,
          B_flat_ref, bbil_ref, WgrnT_ref, gbias_ref, WrankT_ref, brank_ref,
          out_ref, u_ref):
    ent = jnp.dot(ent_ref[...], WentT_ref[...],
                  preferred_element_type=jnp.float32) + bentP_ref[...]
    q = q_ref[...]
    for i in range(D):
        u_ref[:, i * DP:(i + 1) * DP] = q[:, i:i + 1] * ent
    node64 = jnp.dot(u_ref[...], B_flat_ref[...],
                     preferred_element_type=jnp.float32)
    node = node64[:, :D] + bbil_ref[...]

    score = score_ref[...]
    acc = score[:, K_NB:K_NB + 1] * node
    for k in range(K_NB):
        acc = acc + score[:, k:k + 1] * para_ref[:, k, :]
    out_nodes = jnp.dot(acc, WgrnT_ref[...],
                        preferred_element_type=jnp.float32) + gbias_ref[...]
    out_nodes = jnp.where(out_nodes > 0, out_nodes,
                          jnp.exp(jnp.minimum(out_nodes, 0.0)) - 1.0)
    out_ref[...] = jnp.dot(out_nodes, WrankT_ref[...],
                           preferred_element_type=jnp.float32) + brank_ref[...]


@jax.jit
def kernel(query_emb, entity_emb, neighbors_para, neighbors_score, W_ent,
           b_ent, B_bil, b_bil, W_grn, grn_bias, W_rank, b_rank):
    WentT = jnp.zeros((D_ENT, DP), jnp.float32).at[:, :D].set(W_ent.T)
    bentP = jnp.pad(b_ent, (0, DP - D))[None, :]
    B_flat = jnp.transpose(B_bil, (1, 2, 0))
    B_flat = jnp.pad(B_flat, ((0, 0), (0, DP - D), (0, DP - D)))
    B_flat = B_flat.reshape(D * DP, DP)
    WgrnT = W_grn.T
    WrankT = W_rank.T

    grid = (N_NODES // BLK,)
    out = pl.pallas_call(
        _body,
        grid=grid,
        in_specs=[
            pl.BlockSpec((BLK, D), lambda i: (i, 0)),
            pl.BlockSpec((BLK, D_ENT), lambda i: (i, 0)),
            pl.BlockSpec((BLK, K_NB, D), lambda i: (i, 0, 0)),
            pl.BlockSpec((BLK, K_NB + 1), lambda i: (i, 0)),
            pl.BlockSpec((D_ENT, DP), lambda i: (0, 0)),
            pl.BlockSpec((1, DP), lambda i: (0, 0)),
            pl.BlockSpec((D * DP, DP), lambda i: (0, 0)),
            pl.BlockSpec((1, D), lambda i: (0, 0)),
            pl.BlockSpec((D, D), lambda i: (0, 0)),
            pl.BlockSpec((1, D), lambda i: (0, 0)),
            pl.BlockSpec((D, 1), lambda i: (0, 0)),
            pl.BlockSpec((1, 1), lambda i: (0, 0)),
        ],
        out_specs=pl.BlockSpec((BLK, 1), lambda i: (i, 0)),
        out_shape=jax.ShapeDtypeStruct((N_NODES, 1), jnp.float32),
        scratch_shapes=[pltpu.VMEM((BLK, D * DP), jnp.float32)],
        compiler_params=pltpu.CompilerParams(
            dimension_semantics=("parallel",)),
    )(query_emb, entity_emb, neighbors_para, neighbors_score,
      WentT, bentP, B_flat, b_bil[None, :], WgrnT, grn_bias[None, :], WrankT,
      b_rank[None, :])
    return out
